# baseline (speedup 1.0000x reference)
import jax
import jax.numpy as jnp
from jax import lax
from jax.experimental import pallas as pl
from jax.experimental.pallas import tpu as pltpu
from jax.experimental.pallas import tpu_sc as plsc

_N = 50000
_E = 800000
_NPAD = 50176
_RPT = _NPAD // 16
_DUMMY = _N
_W = 2000
_NB = 25
_BB = _W // _NB
_NWIN = _E // (16 * _W)
_WSPLIT = 13

_mesh = plsc.VectorSubcoreMesh(
    core_axis_name="c", subcore_axis_name="s", num_cores=2, num_subcores=16)

_f32 = jnp.float32
_i32 = jnp.int32


def _iota16():
    return lax.broadcasted_iota(_i32, (16,), 0)


def _zero1d(buf, n):
    def z(i, carry):
        buf[pl.ds(i * 16, 16)] = jnp.zeros((16,), _f32)
        return carry
    lax.fori_loop(0, n // 16, z, 0)


def _zero2d(buf, n):
    def z(i, carry):
        buf[i, pl.ds(0, 16)] = jnp.zeros((16,), _f32)
        buf[i, pl.ds(16, 16)] = jnp.zeros((16,), _f32)
        return carry
    lax.fori_loop(0, n, z, 0)


def _sca_body(row_h, col_h, deg0_h, deg1_h, colp_h,
              acc, wbuf, rbuf, cbuf, rp80, cpbuf, ones):
    c = lax.axis_index("c")
    s = lax.axis_index("s")
    _zero1d(wbuf, _RPT)
    pltpu.sync_copy(wbuf, acc.at[pl.ds(s * _RPT, _RPT)])
    for q in range(_BB // 16):
        ones[pl.ds(q * 16, 16)] = jnp.ones((16,), _f32)
    plsc.subcore_barrier()

    wlo = jnp.where(c == 0, 0, _WSPLIT)
    whi = jnp.where(c == 0, _WSPLIT, _NWIN)

    def win(w, carry):
        ebase = s * (_E // 16) + w * _W
        pltpu.sync_copy(row_h.at[pl.ds(ebase, _W)], rbuf)
        pltpu.sync_copy(col_h.at[pl.ds(ebase, _W)], cbuf)

        def batch(j, carry2):
            for q in range(_BB // 16):
                off = j * _BB + q * 16
                rv = rbuf[pl.ds(off, 16)]
                cv = cbuf[pl.ds(off, 16)]
                keep = rv != cv
                rp80[pl.ds(q * 16, 16)] = jnp.where(keep, rv, _DUMMY)
                cpbuf[pl.ds(off, 16)] = jnp.where(keep, cv, _DUMMY)
            pltpu.sync_copy(ones, acc.at[rp80], add=True)
            return carry2
        lax.fori_loop(0, _NB, batch, 0)

        pltpu.sync_copy(cpbuf, colp_h.at[pl.ds(ebase, _W)])
        return carry
    lax.fori_loop(wlo, whi, win, 0)

    plsc.subcore_barrier()
    pltpu.sync_copy(acc.at[pl.ds(s * _RPT, _RPT)], wbuf)

    @pl.when(c == 0)
    def _():
        pltpu.sync_copy(wbuf, deg0_h.at[pl.ds(s * _RPT, _RPT)])

    @pl.when(c == 1)
    def _():
        pltpu.sync_copy(wbuf, deg1_h.at[pl.ds(s * _RPT, _RPT)])


def _sca(row, col):
    k = pl.kernel(
        _sca_body,
        out_type=[jax.ShapeDtypeStruct((_NPAD,), _f32),
                  jax.ShapeDtypeStruct((_NPAD,), _f32),
                  jax.ShapeDtypeStruct((_E,), _i32)],
        mesh=_mesh,
        compiler_params=pltpu.CompilerParams(use_tc_tiling_on_sc=False, needs_layout_passes=False, disable_bounds_checks=True),
        scratch_types=[
            pltpu.VMEM_SHARED((_NPAD,), _f32),
            pltpu.VMEM((_RPT,), _f32),
            pltpu.VMEM((_W,), _i32),
            pltpu.VMEM((_W,), _i32),
            pltpu.VMEM((_BB,), _i32),
            pltpu.VMEM((_W,), _i32),
            pltpu.VMEM((_BB,), _f32),
        ],
    )
    return k(row, col)


def _fill_idx(dst80, src, j):
    for q in range(_BB // 16):
        dst80[pl.ds(q * 16, 16)] = src[pl.ds(j * _BB + q * 16, 16)]


_WCH = _RPT // 8


def _segsum_pass(tab, ti, row_h, colp_h, out_h, acc, wbuf,
                 rbuf, cbuf, cia, cib, rows_a, rows_b,
                 sem_a, sem_b, s):
    _zero2d(wbuf, _WCH)
    for t in range(8):
        pltpu.sync_copy(wbuf, acc.at[pl.ds(s * _RPT + t * _WCH, _WCH)])
    plsc.subcore_barrier()

    def gat(j, dst, sem):
        return pltpu.make_async_copy(
            tab.at[rbuf.at[pl.ds(j * _BB, _BB)]], dst, sem)

    def win(w, carry):
        ebase = s * (_E // 16) + w * _W
        pltpu.sync_copy(row_h.at[pl.ds(ebase, _W)], rbuf)
        pltpu.sync_copy(colp_h.at[pl.ds(ebase, _W)], cbuf)

        gat(0, rows_a, sem_a).start()

        def pair(i, carry2):
            j = 2 * i
            gat(j, rows_a, sem_a).wait()
            gat(j + 1, rows_b, sem_b).start()
            _fill_idx(cia, cbuf, j)
            pltpu.sync_copy(rows_a, acc.at[cia], add=True)

            gat(j + 1, rows_b, sem_b).wait()
            gat(j + 2, rows_a, sem_a).start()
            _fill_idx(cib, cbuf, j + 1)
            pltpu.sync_copy(rows_b, acc.at[cib], add=True)
            return carry2
        lax.fori_loop(0, (_NB - 1) // 2, pair, 0)

        gat(_NB - 1, rows_a, sem_a).wait()
        _fill_idx(cia, cbuf, _NB - 1)
        pltpu.sync_copy(rows_a, acc.at[cia], add=True)
        return carry
    lax.fori_loop(0, _NWIN, win, 0)

    plsc.subcore_barrier()
    for t in range(8):
        pltpu.sync_copy(acc.at[pl.ds(s * _RPT + t * _WCH, _WCH)], wbuf)
        pltpu.sync_copy(wbuf, out_h.at[ti, pl.ds(s * _RPT + t * _WCH, _WCH)])


def _make_segsum(nt):
    tpc = nt // 2

    def body(*refs):
        row_h, colp_h = refs[0], refs[1]
        tabs = refs[2:2 + nt]
        out_h = refs[2 + nt]
        (acc, wbuf, rbuf, cbuf, cia, cib,
         rows_a, rows_b, sem_a, sem_b) = refs[3 + nt:]
        c = lax.axis_index("c")
        s = lax.axis_index("s")
        for cc in range(2):
            @pl.when(c == cc)
            def _():
                for j in range(tpc):
                    ti = cc * tpc + j
                    _segsum_pass(tabs[ti], ti, row_h, colp_h,
                                 out_h, acc, wbuf, rbuf, cbuf, cia, cib,
                                 rows_a, rows_b, sem_a, sem_b, s)

    def run(row, colp, *tables):
        k = pl.kernel(
            body,
            out_type=[jax.ShapeDtypeStruct((nt, _NPAD, 32), _f32)],
            mesh=_mesh,
            compiler_params=pltpu.CompilerParams(use_tc_tiling_on_sc=False, needs_layout_passes=False, disable_bounds_checks=True),
            scratch_types=[
                pltpu.VMEM_SHARED((_NPAD, 32), _f32),
                pltpu.VMEM((_WCH, 32), _f32),
                pltpu.VMEM((_W,), _i32),
                pltpu.VMEM((_W,), _i32),
                pltpu.VMEM((_BB,), _i32),
                pltpu.VMEM((_BB,), _i32),
                pltpu.VMEM((_BB, 32), _f32),
                pltpu.VMEM((_BB, 32), _f32),
                pltpu.SemaphoreType.DMA,
                pltpu.SemaphoreType.DMA,
            ],
        )
        return k(row, colp, *tables)
    return run


_segsum2 = _make_segsum(2)
_segsum4 = _make_segsum(4)


_DZ = 112
_DW = 64


def _scd_body(row_h, col_h, colp_h, nrow_h, ncol_h, zt_h, ys_h,
              posp_h, negp_h, sy0_h, sy1_h,
              acc1, wbuf, rbuf, cbuf, cpbuf, cia, zra, zca, zrb, zcb,
              yrows, pbuf, sem_a, sem_b, sem_c, sem_d, sem_y):
    c = lax.axis_index("c")
    s = lax.axis_index("s")
    _zero1d(wbuf, _RPT)
    pltpu.sync_copy(wbuf, acc1.at[pl.ds(s * _RPT, _RPT)])
    plsc.subcore_barrier()

    wlo = jnp.where(c == 0, 0, _WSPLIT)
    whi = jnp.where(c == 0, _WSPLIT, _NWIN)

    def gz(idxbuf, j, dst, sem):
        return pltpu.make_async_copy(
            zt_h.at[idxbuf.at[pl.ds(j * _BB, _BB)]], dst, sem)

    def win(w, carry):
        ebase = s * (_E // 16) + w * _W
        pltpu.sync_copy(colp_h.at[pl.ds(ebase, _W)], cpbuf)

        def phase(p, carry1):
            @pl.when(p == 0)
            def _():
                pltpu.sync_copy(row_h.at[pl.ds(ebase, _W)], rbuf)
                pltpu.sync_copy(col_h.at[pl.ds(ebase, _W)], cbuf)

            @pl.when(p == 1)
            def _():
                pltpu.sync_copy(nrow_h.at[pl.ds(ebase, _W)], rbuf)
                pltpu.sync_copy(ncol_h.at[pl.ds(ebase, _W)], cbuf)

            def step(j, zr, zc, sr, sc2, nzr, nzc, nsr, nsc, issue_next):
                gz(rbuf, j, zr, sr).wait()
                gz(cbuf, j, zc, sc2).wait()
                if issue_next:
                    gz(rbuf, j + 1, nzr, nsr).start()
                    gz(cbuf, j + 1, nzc, nsc).start()

                @pl.when(p == 0)
                def _():
                    _fill_idx(cia, cpbuf, j)
                    pltpu.async_copy(
                        ys_h.at[rbuf.at[pl.ds(j * _BB, _BB)]],
                        yrows, sem_y).wait()
                    pltpu.sync_copy(yrows, acc1.at[cia], add=True)

                def chunk(k, carry2):
                    eidx = k * 16 + _iota16()
                    accs = [jnp.zeros((16,), _f32) for _ in range(4)]
                    for f in range(_DZ):
                        fidx = jnp.full((16,), f, _i32)
                        accs[f % 4] = accs[f % 4] + (
                            plsc.load_gather(zr, [eidx, fidx]) *
                            plsc.load_gather(zc, [eidx, fidx]))
                    pbuf[pl.ds(j * _BB + k * 16, 16)] = (
                        (accs[0] + accs[1]) + (accs[2] + accs[3]))
                    return carry2
                lax.fori_loop(0, _BB // 16, chunk, 0)

            gz(rbuf, 0, zra, sem_a).start()
            gz(cbuf, 0, zca, sem_b).start()

            def pair(i, carry2):
                j = 2 * i
                step(j, zra, zca, sem_a, sem_b,
                     zrb, zcb, sem_c, sem_d, True)
                step(j + 1, zrb, zcb, sem_c, sem_d,
                     zra, zca, sem_a, sem_b, True)
                return carry2
            lax.fori_loop(0, (_NB - 1) // 2, pair, 0)
            step(_NB - 1, zra, zca, sem_a, sem_b, None, None, None, None,
                 False)

            @pl.when(p == 0)
            def _():
                pltpu.sync_copy(pbuf, posp_h.at[pl.ds(ebase, _W)])

            @pl.when(p == 1)
            def _():
                pltpu.sync_copy(pbuf, negp_h.at[pl.ds(ebase, _W)])
            return carry1
        lax.fori_loop(0, 2, phase, 0)
        return carry
    lax.fori_loop(wlo, whi, win, 0)

    plsc.subcore_barrier()
    pltpu.sync_copy(acc1.at[pl.ds(s * _RPT, _RPT)], wbuf)

    @pl.when(c == 0)
    def _():
        pltpu.sync_copy(wbuf, sy0_h.at[pl.ds(s * _RPT, _RPT)])

    @pl.when(c == 1)
    def _():
        pltpu.sync_copy(wbuf, sy1_h.at[pl.ds(s * _RPT, _RPT)])


def _scd(row, col, colp, nrow, ncol, zt, ys):
    k = pl.kernel(
        _scd_body,
        out_type=[jax.ShapeDtypeStruct((_E,), _f32),
                  jax.ShapeDtypeStruct((_E,), _f32),
                  jax.ShapeDtypeStruct((_NPAD,), _f32),
                  jax.ShapeDtypeStruct((_NPAD,), _f32)],
        mesh=_mesh,
        compiler_params=pltpu.CompilerParams(use_tc_tiling_on_sc=False, needs_layout_passes=False, disable_bounds_checks=True),
        scratch_types=[
            pltpu.VMEM_SHARED((_NPAD,), _f32),
            pltpu.VMEM((_RPT,), _f32),
            pltpu.VMEM((_W,), _i32),
            pltpu.VMEM((_W,), _i32),
            pltpu.VMEM((_W,), _i32),
            pltpu.VMEM((_BB,), _i32),
            pltpu.VMEM((_BB, _DZ), _f32),
            pltpu.VMEM((_BB, _DZ), _f32),
            pltpu.VMEM((_BB, _DZ), _f32),
            pltpu.VMEM((_BB, _DZ), _f32),
            pltpu.VMEM((_BB,), _f32),
            pltpu.VMEM((_W,), _f32),
            pltpu.SemaphoreType.DMA,
            pltpu.SemaphoreType.DMA,
            pltpu.SemaphoreType.DMA,
            pltpu.SemaphoreType.DMA,
            pltpu.SemaphoreType.DMA,
        ],
    )
    return k(row, col, colp, nrow, ncol, zt, ys)


_BLK = 1000
_NBLK = _N // _BLK


def _full(shape):
    return pl.BlockSpec(shape, lambda i: tuple(0 for _ in shape))


def _rows(width):
    return pl.BlockSpec((_BLK, width), lambda i: (i, 0))


def _tca_body(x_ref, d0_ref, d1_ref, w0_ref, l1w_ref, l1b_ref, l2w_ref,
              l2b_ref,
              xs0_ref, xs1_ref, xw0_ref, l1_ref, l2_ref, dis_ref):
    x = x_ref[...]
    deg = d0_ref[...] + d1_ref[...]
    dis = jnp.where(deg > 0.0, lax.rsqrt(jnp.maximum(deg, 1e-12)), 0.0)
    dis_ref[...] = dis
    xs = x * dis
    xs0_ref[...] = xs[:, :32]
    xs1_ref[...] = jnp.concatenate(
        [xs[:, 32:], jnp.zeros((_BLK, 6), _f32)], axis=1)
    xw0_ref[...] = jnp.dot(x, w0_ref[...], preferred_element_type=_f32)
    l1_ref[...] = jax.nn.relu(
        jnp.dot(x, l1w_ref[...], preferred_element_type=_f32) + l1b_ref[...])
    l2_ref[...] = jax.nn.relu(
        jnp.dot(x, l2w_ref[...], preferred_element_type=_f32) + l2b_ref[...])


def _tca(x, deg0, deg1, w0, l1w, l1b, l2w, l2b):
    return pl.pallas_call(
        _tca_body,
        grid=(_NBLK,),
        in_specs=[_rows(58), _rows(1), _rows(1),
                  _full((58, 300)), _full((58, 100)), _full((1, 100)),
                  _full((58, 100)), _full((1, 100))],
        out_specs=[_rows(32), _rows(32), _rows(300), _rows(100), _rows(100),
                   _rows(1)],
        out_shape=[jax.ShapeDtypeStruct((_N, 32), _f32),
                   jax.ShapeDtypeStruct((_N, 32), _f32),
                   jax.ShapeDtypeStruct((_N, 300), _f32),
                   jax.ShapeDtypeStruct((_N, 100), _f32),
                   jax.ShapeDtypeStruct((_N, 100), _f32),
                   jax.ShapeDtypeStruct((_N, 1), _f32)],
    )(x, deg0, deg1, w0, l1w, l1b, l2w, l2b)


def _tcb_body(xw0_ref, sx_ref, dis_ref, w1a_ref, w1b_ref, b1_ref,
              c2w0_ref, c2w1_ref,
              hw0_ref, hs0_ref, hs1_ref, hs2_ref, hs3_ref):
    sx = sx_ref[...]
    dis = dis_ref[...]
    tx1 = (jnp.dot(sx[0], w1a_ref[...], preferred_element_type=_f32) +
           jnp.dot(sx[1], w1b_ref[...], preferred_element_type=_f32))
    h = jax.nn.relu(xw0_ref[...] - dis * tx1 + b1_ref[...])
    hw0_ref[...] = jnp.dot(h, c2w0_ref[...], preferred_element_type=_f32)
    hs = dis * jnp.dot(h, c2w1_ref[...], preferred_element_type=_f32)
    hs0_ref[...] = hs[:, :32]
    hs1_ref[...] = hs[:, 32:64]
    hs2_ref[...] = hs[:, 64:96]
    hs3_ref[...] = jnp.concatenate(
        [hs[:, 96:], jnp.zeros((_BLK, 28), _f32)], axis=1)


def _tcb(xw0, sx, dis, w1a, w1b, b1, c2w0, c2w1):
    return pl.pallas_call(
        _tcb_body,
        grid=(_NBLK,),
        in_specs=[_rows(300),
                  pl.BlockSpec((2, _BLK, 32), lambda i: (0, i, 0)),
                  _rows(1),
                  _full((32, 300)), _full((32, 300)), _full((1, 300)),
                  _full((300, 100)), _full((300, 100))],
        out_specs=[_rows(100), _rows(32), _rows(32), _rows(32), _rows(32)],
        out_shape=[jax.ShapeDtypeStruct((_N, 100), _f32)] +
                  [jax.ShapeDtypeStruct((_N, 32), _f32)] * 4,
    )(xw0, sx, dis, w1a, w1b, b1, c2w0, c2w1)


def _tcc_body(hw0_ref, shw_ref, dis_ref, b2_ref, l1_ref, l2_ref, c3w1_ref,
              xm_ref, zt_ref, ys_ref):
    shw = shw_ref[...]
    dis = dis_ref[...]
    svals = jnp.concatenate([shw[0], shw[1], shw[2], shw[3]], axis=1)[:, :100]
    x1 = jax.nn.relu(hw0_ref[...] - dis * svals + b2_ref[...])
    xm = x1 + l1_ref[...]
    z = x1 + l2_ref[...]
    xm_ref[...] = xm
    zt_ref[...] = jnp.concatenate(
        [z, jnp.zeros((_BLK, _DZ - 100), _f32)], axis=1)
    ys_ref[...] = dis * jnp.dot(xm, c3w1_ref[...],
                                preferred_element_type=_f32)


def _tcc(hw0, shw, dis, b2, l1, l2, c3w1):
    return pl.pallas_call(
        _tcc_body,
        grid=(_NBLK,),
        in_specs=[_rows(100),
                  pl.BlockSpec((4, _BLK, 32), lambda i: (0, i, 0)),
                  _rows(1), _full((1, 100)), _rows(100), _rows(100),
                  _full((100, 1))],
        out_specs=[_rows(100), _rows(_DZ), _rows(1)],
        out_shape=[jax.ShapeDtypeStruct((_N, 100), _f32),
                   jax.ShapeDtypeStruct((_N, _DZ), _f32),
                   jax.ShapeDtypeStruct((_N, 1), _f32)],
    )(hw0, shw, dis, b2, l1, l2, c3w1)


def _tcd_body(xm_ref, c3w0_ref, b3_ref, dis_ref, sy0_ref, sy1_ref, out_ref):
    sy = sy0_ref[...] + sy1_ref[...]
    out_ref[...] = (jnp.dot(xm_ref[...], c3w0_ref[...],
                            preferred_element_type=_f32)
                    - dis_ref[...] * sy + b3_ref[...])


def _tcd(xm, c3w0, b3, dis, sy0, sy1):
    return pl.pallas_call(
        _tcd_body,
        grid=(_NBLK,),
        in_specs=[_rows(100), _full((100, 1)), _full((1, 1)), _rows(1),
                  _rows(1), _rows(1)],
        out_specs=_rows(1),
        out_shape=jax.ShapeDtypeStruct((_N, 1), _f32),
    )(xm, c3w0, b3, dis, sy0, sy1)


_LBLK = 6400
_LNBLK = _E // _LBLK


def _loss_body(p_ref, n_ref, out_ref, acc_ref):
    i = pl.program_id(0)

    @pl.when(i == 0)
    def _():
        acc_ref[0, 0] = 0.0

    sp = p_ref[0]
    sn = n_ref[0]
    term = (jnp.sum(jnp.log(jax.nn.sigmoid(sp) + 1e-15)) +
            jnp.sum(jnp.log(1.0 - jax.nn.sigmoid(sn))))
    acc_ref[0, 0] += term

    @pl.when(i == _LNBLK - 1)
    def _():
        out_ref[...] = jnp.full((1, 1), -acc_ref[0, 0] / float(_E), _f32)


def _loss(p, n):
    espec = pl.BlockSpec((1, _LBLK), lambda i: (0, i))
    return pl.pallas_call(
        _loss_body,
        grid=(_LNBLK,),
        in_specs=[espec, espec],
        out_specs=pl.BlockSpec((1, 1), lambda i: (0, 0)),
        out_shape=jax.ShapeDtypeStruct((1, 1), _f32),
        scratch_shapes=[pltpu.SMEM((1, 1), _f32)],
    )(p, n)


def kernel(x, edge_index, conv1_W0, conv1_W1, conv1_b, conv2_W0, conv2_W1,
           conv2_b, conv3_W0, conv3_W1, conv3_b, lin1_W, lin1_b, lin2_W,
           lin2_b, c1, c2):
    row = edge_index[0]
    col = edge_index[1]
    neg_ei = jax.random.randint(jax.random.key(1), (2, _E), 0, _N,
                                dtype=jnp.int32)

    deg0, deg1, colp = _sca(row, col)

    b1 = conv1_b.reshape(1, 300)
    l1b = lin1_b.reshape(1, 100)
    l2b = lin2_b.reshape(1, 100)
    xs0, xs1, xw0, l1, l2, dis = _tca(x, deg0[:_N, None], deg1[:_N, None],
                                      conv1_W0, lin1_W, l1b, lin2_W, l2b)

    (sx,) = _segsum2(row, colp, xs0, xs1)

    w1pad = jnp.concatenate([conv1_W1, jnp.zeros((6, 300), _f32)], axis=0)
    hw0, hs0, hs1, hs2, hs3 = _tcb(xw0, sx[:, :_N, :], dis, w1pad[:32],
                                   w1pad[32:], b1, conv2_W0, conv2_W1)

    (shw,) = _segsum4(row, colp, hs0, hs1, hs2, hs3)

    b2 = conv2_b.reshape(1, 100)
    xm, zt, ys = _tcc(hw0, shw[:, :_N, :], dis, b2, l1, l2, conv3_W1)

    posp, negp, sy0, sy1 = _scd(row, col, colp, neg_ei[0], neg_ei[1],
                                zt, ys.reshape(_N))

    b3 = conv3_b.reshape(1, 1)
    out = _tcd(xm, conv3_W0, b3, dis, sy0[:_N, None], sy1[:_N, None])
    r_loss = _loss(posp.reshape(1, _E), negp.reshape(1, _E)).reshape(())
    return (out, r_loss, c1, c2)


# speedup vs baseline: 1.1815x; 1.1815x over previous
import jax
import jax.numpy as jnp
from jax import lax
from jax.experimental import pallas as pl
from jax.experimental.pallas import tpu as pltpu
from jax.experimental.pallas import tpu_sc as plsc

_N = 50000
_E = 800000
_NPAD = 50176
_RPT = _NPAD // 16
_DUMMY = _N
_W = 2000
_NB = 25
_BB = _W // _NB
_NWIN = _E // (16 * _W)
_WSPLIT = 13

_mesh = plsc.VectorSubcoreMesh(
    core_axis_name="c", subcore_axis_name="s", num_cores=2, num_subcores=16)

_f32 = jnp.float32
_i32 = jnp.int32


def _iota16():
    return lax.broadcasted_iota(_i32, (16,), 0)


def _zero1d(buf, n):
    def z(i, carry):
        buf[pl.ds(i * 16, 16)] = jnp.zeros((16,), _f32)
        return carry
    lax.fori_loop(0, n // 16, z, 0)


def _zero2d(buf, n):
    def z(i, carry):
        buf[i, pl.ds(0, 16)] = jnp.zeros((16,), _f32)
        buf[i, pl.ds(16, 16)] = jnp.zeros((16,), _f32)
        return carry
    lax.fori_loop(0, n, z, 0)


def _sca_body(row_h, col_h, deg0_h, deg1_h, colp_h,
              acc, wbuf, rbuf, cbuf, rp80, cpbuf, ones):
    c = lax.axis_index("c")
    s = lax.axis_index("s")
    _zero1d(wbuf, _RPT)
    pltpu.sync_copy(wbuf, acc.at[pl.ds(s * _RPT, _RPT)])
    for q in range(_BB // 16):
        ones[pl.ds(q * 16, 16)] = jnp.ones((16,), _f32)
    plsc.subcore_barrier()

    wlo = jnp.where(c == 0, 0, _WSPLIT)
    whi = jnp.where(c == 0, _WSPLIT, _NWIN)

    def win(w, carry):
        ebase = s * (_E // 16) + w * _W
        pltpu.sync_copy(row_h.at[pl.ds(ebase, _W)], rbuf)
        pltpu.sync_copy(col_h.at[pl.ds(ebase, _W)], cbuf)

        def batch(j, carry2):
            for q in range(_BB // 16):
                off = j * _BB + q * 16
                rv = rbuf[pl.ds(off, 16)]
                cv = cbuf[pl.ds(off, 16)]
                keep = rv != cv
                rp80[pl.ds(q * 16, 16)] = jnp.where(keep, rv, _DUMMY)
                cpbuf[pl.ds(off, 16)] = jnp.where(keep, cv, _DUMMY)
            pltpu.sync_copy(ones, acc.at[rp80], add=True)
            return carry2
        lax.fori_loop(0, _NB, batch, 0)

        pltpu.sync_copy(cpbuf, colp_h.at[pl.ds(ebase, _W)])
        return carry
    lax.fori_loop(wlo, whi, win, 0)

    plsc.subcore_barrier()
    pltpu.sync_copy(acc.at[pl.ds(s * _RPT, _RPT)], wbuf)

    @pl.when(c == 0)
    def _():
        pltpu.sync_copy(wbuf, deg0_h.at[pl.ds(s * _RPT, _RPT)])

    @pl.when(c == 1)
    def _():
        pltpu.sync_copy(wbuf, deg1_h.at[pl.ds(s * _RPT, _RPT)])


def _sca(row, col):
    k = pl.kernel(
        _sca_body,
        out_type=[jax.ShapeDtypeStruct((_NPAD,), _f32),
                  jax.ShapeDtypeStruct((_NPAD,), _f32),
                  jax.ShapeDtypeStruct((_E,), _i32)],
        mesh=_mesh,
        compiler_params=pltpu.CompilerParams(use_tc_tiling_on_sc=False, needs_layout_passes=False, disable_bounds_checks=True),
        scratch_types=[
            pltpu.VMEM_SHARED((_NPAD,), _f32),
            pltpu.VMEM((_RPT,), _f32),
            pltpu.VMEM((_W,), _i32),
            pltpu.VMEM((_W,), _i32),
            pltpu.VMEM((_BB,), _i32),
            pltpu.VMEM((_W,), _i32),
            pltpu.VMEM((_BB,), _f32),
        ],
    )
    return k(row, col)


def _fill_idx(dst80, src, j):
    for q in range(_BB // 16):
        dst80[pl.ds(q * 16, 16)] = src[pl.ds(j * _BB + q * 16, 16)]


_WCH = _RPT // 8


def _segsum_pass(tab, ti, row_h, colp_h, out_h, acc, wbuf,
                 rbuf, cbuf, cia, cib, rows_a, rows_b,
                 sem_a, sem_b, s):
    _zero2d(wbuf, _WCH)
    for t in range(8):
        pltpu.sync_copy(wbuf, acc.at[pl.ds(s * _RPT + t * _WCH, _WCH)])
    plsc.subcore_barrier()

    def gat(j, dst, sem):
        return pltpu.make_async_copy(
            tab.at[rbuf.at[pl.ds(j * _BB, _BB)]], dst, sem)

    def win(w, carry):
        ebase = s * (_E // 16) + w * _W
        pltpu.sync_copy(row_h.at[pl.ds(ebase, _W)], rbuf)
        pltpu.sync_copy(colp_h.at[pl.ds(ebase, _W)], cbuf)

        gat(0, rows_a, sem_a).start()

        def pair(i, carry2):
            j = 2 * i
            gat(j, rows_a, sem_a).wait()
            gat(j + 1, rows_b, sem_b).start()
            _fill_idx(cia, cbuf, j)
            pltpu.sync_copy(rows_a, acc.at[cia], add=True)

            gat(j + 1, rows_b, sem_b).wait()
            gat(j + 2, rows_a, sem_a).start()
            _fill_idx(cib, cbuf, j + 1)
            pltpu.sync_copy(rows_b, acc.at[cib], add=True)
            return carry2
        lax.fori_loop(0, (_NB - 1) // 2, pair, 0)

        gat(_NB - 1, rows_a, sem_a).wait()
        _fill_idx(cia, cbuf, _NB - 1)
        pltpu.sync_copy(rows_a, acc.at[cia], add=True)
        return carry
    lax.fori_loop(0, _NWIN, win, 0)

    plsc.subcore_barrier()
    for t in range(8):
        pltpu.sync_copy(acc.at[pl.ds(s * _RPT + t * _WCH, _WCH)], wbuf)
        pltpu.sync_copy(wbuf, out_h.at[ti, pl.ds(s * _RPT + t * _WCH, _WCH)])


def _make_segsum(nt):
    tpc = nt // 2

    def body(*refs):
        row_h, colp_h = refs[0], refs[1]
        tabs = refs[2:2 + nt]
        out_h = refs[2 + nt]
        (acc, wbuf, rbuf, cbuf, cia, cib,
         rows_a, rows_b, sem_a, sem_b) = refs[3 + nt:]
        c = lax.axis_index("c")
        s = lax.axis_index("s")
        for cc in range(2):
            @pl.when(c == cc)
            def _():
                for j in range(tpc):
                    ti = cc * tpc + j
                    _segsum_pass(tabs[ti], ti, row_h, colp_h,
                                 out_h, acc, wbuf, rbuf, cbuf, cia, cib,
                                 rows_a, rows_b, sem_a, sem_b, s)

    def run(row, colp, *tables):
        k = pl.kernel(
            body,
            out_type=[jax.ShapeDtypeStruct((nt, _NPAD, 32), _f32)],
            mesh=_mesh,
            compiler_params=pltpu.CompilerParams(use_tc_tiling_on_sc=False, needs_layout_passes=False, disable_bounds_checks=True),
            scratch_types=[
                pltpu.VMEM_SHARED((_NPAD, 32), _f32),
                pltpu.VMEM((_WCH, 32), _f32),
                pltpu.VMEM((_W,), _i32),
                pltpu.VMEM((_W,), _i32),
                pltpu.VMEM((_BB,), _i32),
                pltpu.VMEM((_BB,), _i32),
                pltpu.VMEM((_BB, 32), _f32),
                pltpu.VMEM((_BB, 32), _f32),
                pltpu.SemaphoreType.DMA,
                pltpu.SemaphoreType.DMA,
            ],
        )
        return k(row, colp, *tables)
    return run


_segsum2 = _make_segsum(2)
_segsum4 = _make_segsum(4)


_DZ = 112
_DW = 64


def _scd_body(row_h, col_h, colp_h, nrow_h, ncol_h, zt_h, ys_h,
              posp_h, negp_h, sy0_h, sy1_h,
              acc1, wbuf, rbuf, cbuf, cpbuf, cia, zra, zca, zrb, zcb,
              yrows, pbuf, sem_a, sem_b, sem_c, sem_d, sem_y):
    c = lax.axis_index("c")
    s = lax.axis_index("s")
    _zero1d(wbuf, _RPT)
    pltpu.sync_copy(wbuf, acc1.at[pl.ds(s * _RPT, _RPT)])
    plsc.subcore_barrier()

    wlo = jnp.where(c == 0, 0, _WSPLIT)
    whi = jnp.where(c == 0, _WSPLIT, _NWIN)

    def gz(idxbuf, j, dst, sem):
        return pltpu.make_async_copy(
            zt_h.at[idxbuf.at[pl.ds(j * _BB, _BB)]], dst, sem)

    def win(w, carry):
        ebase = s * (_E // 16) + w * _W
        pltpu.sync_copy(colp_h.at[pl.ds(ebase, _W)], cpbuf)

        def phase(p, carry1):
            @pl.when(p == 0)
            def _():
                pltpu.sync_copy(row_h.at[pl.ds(ebase, _W)], rbuf)
                pltpu.sync_copy(col_h.at[pl.ds(ebase, _W)], cbuf)

            @pl.when(p == 1)
            def _():
                pltpu.sync_copy(nrow_h.at[pl.ds(ebase, _W)], rbuf)
                pltpu.sync_copy(ncol_h.at[pl.ds(ebase, _W)], cbuf)

            def step(j, zr, zc, sr, sc2, nzr, nzc, nsr, nsc, issue_next):
                gz(rbuf, j, zr, sr).wait()
                gz(cbuf, j, zc, sc2).wait()
                if issue_next:
                    gz(rbuf, j + 1, nzr, nsr).start()
                    gz(cbuf, j + 1, nzc, nsc).start()

                @pl.when(p == 0)
                def _():
                    _fill_idx(cia, cpbuf, j)
                    pltpu.async_copy(
                        ys_h.at[rbuf.at[pl.ds(j * _BB, _BB)]],
                        yrows, sem_y).wait()
                    pltpu.sync_copy(yrows, acc1.at[cia], add=True)

                def chunk(k, carry2):
                    eidx = k * 16 + _iota16()
                    accv = jnp.zeros((16,), _f32)
                    for f in range(_DZ):
                        fidx = jnp.full((16,), f, _i32)
                        accv = accv + (plsc.load_gather(zr, [eidx, fidx]) *
                                       plsc.load_gather(zc, [eidx, fidx]))
                    pbuf[pl.ds(j * _BB + k * 16, 16)] = accv
                    return carry2
                lax.fori_loop(0, _BB // 16, chunk, 0)

            gz(rbuf, 0, zra, sem_a).start()
            gz(cbuf, 0, zca, sem_b).start()

            def pair(i, carry2):
                j = 2 * i
                step(j, zra, zca, sem_a, sem_b,
                     zrb, zcb, sem_c, sem_d, True)
                step(j + 1, zrb, zcb, sem_c, sem_d,
                     zra, zca, sem_a, sem_b, True)
                return carry2
            lax.fori_loop(0, (_NB - 1) // 2, pair, 0)
            step(_NB - 1, zra, zca, sem_a, sem_b, None, None, None, None,
                 False)

            @pl.when(p == 0)
            def _():
                pltpu.sync_copy(pbuf, posp_h.at[pl.ds(ebase, _W)])

            @pl.when(p == 1)
            def _():
                pltpu.sync_copy(pbuf, negp_h.at[pl.ds(ebase, _W)])
            return carry1
        lax.fori_loop(0, 2, phase, 0)
        return carry
    lax.fori_loop(wlo, whi, win, 0)

    plsc.subcore_barrier()
    pltpu.sync_copy(acc1.at[pl.ds(s * _RPT, _RPT)], wbuf)

    @pl.when(c == 0)
    def _():
        pltpu.sync_copy(wbuf, sy0_h.at[pl.ds(s * _RPT, _RPT)])

    @pl.when(c == 1)
    def _():
        pltpu.sync_copy(wbuf, sy1_h.at[pl.ds(s * _RPT, _RPT)])


def _scd(row, col, colp, nrow, ncol, zt, ys):
    k = pl.kernel(
        _scd_body,
        out_type=[jax.ShapeDtypeStruct((_E,), _f32),
                  jax.ShapeDtypeStruct((_E,), _f32),
                  jax.ShapeDtypeStruct((_NPAD,), _f32),
                  jax.ShapeDtypeStruct((_NPAD,), _f32)],
        mesh=_mesh,
        compiler_params=pltpu.CompilerParams(use_tc_tiling_on_sc=False, needs_layout_passes=False, disable_bounds_checks=True),
        scratch_types=[
            pltpu.VMEM_SHARED((_NPAD,), _f32),
            pltpu.VMEM((_RPT,), _f32),
            pltpu.VMEM((_W,), _i32),
            pltpu.VMEM((_W,), _i32),
            pltpu.VMEM((_W,), _i32),
            pltpu.VMEM((_BB,), _i32),
            pltpu.VMEM((_BB, _DZ), _f32),
            pltpu.VMEM((_BB, _DZ), _f32),
            pltpu.VMEM((_BB, _DZ), _f32),
            pltpu.VMEM((_BB, _DZ), _f32),
            pltpu.VMEM((_BB,), _f32),
            pltpu.VMEM((_W,), _f32),
            pltpu.SemaphoreType.DMA,
            pltpu.SemaphoreType.DMA,
            pltpu.SemaphoreType.DMA,
            pltpu.SemaphoreType.DMA,
            pltpu.SemaphoreType.DMA,
        ],
    )
    return k(row, col, colp, nrow, ncol, zt, ys)


_BLK = 1000
_NBLK = _N // _BLK


def _full(shape):
    return pl.BlockSpec(shape, lambda i: tuple(0 for _ in shape))


def _rows(width):
    return pl.BlockSpec((_BLK, width), lambda i: (i, 0))


def _tca_body(x_ref, d0_ref, d1_ref, w0_ref, l1w_ref, l1b_ref, l2w_ref,
              l2b_ref,
              xs0_ref, xs1_ref, xw0_ref, l1_ref, l2_ref, dis_ref):
    x = x_ref[...]
    deg = d0_ref[...] + d1_ref[...]
    dis = jnp.where(deg > 0.0, lax.rsqrt(jnp.maximum(deg, 1e-12)), 0.0)
    dis_ref[...] = dis
    xs = x * dis
    xs0_ref[...] = xs[:, :32]
    xs1_ref[...] = jnp.concatenate(
        [xs[:, 32:], jnp.zeros((_BLK, 6), _f32)], axis=1)
    xw0_ref[...] = jnp.dot(x, w0_ref[...], preferred_element_type=_f32)
    l1_ref[...] = jax.nn.relu(
        jnp.dot(x, l1w_ref[...], preferred_element_type=_f32) + l1b_ref[...])
    l2_ref[...] = jax.nn.relu(
        jnp.dot(x, l2w_ref[...], preferred_element_type=_f32) + l2b_ref[...])


def _tca(x, deg0, deg1, w0, l1w, l1b, l2w, l2b):
    return pl.pallas_call(
        _tca_body,
        grid=(_NBLK,),
        in_specs=[_rows(58), _rows(1), _rows(1),
                  _full((58, 300)), _full((58, 100)), _full((1, 100)),
                  _full((58, 100)), _full((1, 100))],
        out_specs=[_rows(32), _rows(32), _rows(300), _rows(100), _rows(100),
                   _rows(1)],
        out_shape=[jax.ShapeDtypeStruct((_N, 32), _f32),
                   jax.ShapeDtypeStruct((_N, 32), _f32),
                   jax.ShapeDtypeStruct((_N, 300), _f32),
                   jax.ShapeDtypeStruct((_N, 100), _f32),
                   jax.ShapeDtypeStruct((_N, 100), _f32),
                   jax.ShapeDtypeStruct((_N, 1), _f32)],
    )(x, deg0, deg1, w0, l1w, l1b, l2w, l2b)


def _tcb_body(xw0_ref, sx_ref, dis_ref, w1a_ref, w1b_ref, b1_ref,
              c2w0_ref, c2w1_ref,
              hw0_ref, hs0_ref, hs1_ref, hs2_ref, hs3_ref):
    sx = sx_ref[...]
    dis = dis_ref[...]
    tx1 = (jnp.dot(sx[0], w1a_ref[...], preferred_element_type=_f32) +
           jnp.dot(sx[1], w1b_ref[...], preferred_element_type=_f32))
    h = jax.nn.relu(xw0_ref[...] - dis * tx1 + b1_ref[...])
    hw0_ref[...] = jnp.dot(h, c2w0_ref[...], preferred_element_type=_f32)
    hs = dis * jnp.dot(h, c2w1_ref[...], preferred_element_type=_f32)
    hs0_ref[...] = hs[:, :32]
    hs1_ref[...] = hs[:, 32:64]
    hs2_ref[...] = hs[:, 64:96]
    hs3_ref[...] = jnp.concatenate(
        [hs[:, 96:], jnp.zeros((_BLK, 28), _f32)], axis=1)


def _tcb(xw0, sx, dis, w1a, w1b, b1, c2w0, c2w1):
    return pl.pallas_call(
        _tcb_body,
        grid=(_NBLK,),
        in_specs=[_rows(300),
                  pl.BlockSpec((2, _BLK, 32), lambda i: (0, i, 0)),
                  _rows(1),
                  _full((32, 300)), _full((32, 300)), _full((1, 300)),
                  _full((300, 100)), _full((300, 100))],
        out_specs=[_rows(100), _rows(32), _rows(32), _rows(32), _rows(32)],
        out_shape=[jax.ShapeDtypeStruct((_N, 100), _f32)] +
                  [jax.ShapeDtypeStruct((_N, 32), _f32)] * 4,
    )(xw0, sx, dis, w1a, w1b, b1, c2w0, c2w1)


def _tcc_body(hw0_ref, shw_ref, dis_ref, b2_ref, l1_ref, l2_ref, c3w1_ref,
              xm_ref, zt_ref, ys_ref):
    shw = shw_ref[...]
    dis = dis_ref[...]
    svals = jnp.concatenate([shw[0], shw[1], shw[2], shw[3]], axis=1)[:, :100]
    x1 = jax.nn.relu(hw0_ref[...] - dis * svals + b2_ref[...])
    xm = x1 + l1_ref[...]
    z = x1 + l2_ref[...]
    xm_ref[...] = xm
    zt_ref[...] = jnp.concatenate(
        [z, jnp.zeros((_BLK, _DZ - 100), _f32)], axis=1)
    ys_ref[...] = dis * jnp.dot(xm, c3w1_ref[...],
                                preferred_element_type=_f32)


def _tcc(hw0, shw, dis, b2, l1, l2, c3w1):
    return pl.pallas_call(
        _tcc_body,
        grid=(_NBLK,),
        in_specs=[_rows(100),
                  pl.BlockSpec((4, _BLK, 32), lambda i: (0, i, 0)),
                  _rows(1), _full((1, 100)), _rows(100), _rows(100),
                  _full((100, 1))],
        out_specs=[_rows(100), _rows(_DZ), _rows(1)],
        out_shape=[jax.ShapeDtypeStruct((_N, 100), _f32),
                   jax.ShapeDtypeStruct((_N, _DZ), _f32),
                   jax.ShapeDtypeStruct((_N, 1), _f32)],
    )(hw0, shw, dis, b2, l1, l2, c3w1)


def _tcd_body(xm_ref, c3w0_ref, b3_ref, dis_ref, sy0_ref, sy1_ref, out_ref):
    sy = sy0_ref[...] + sy1_ref[...]
    out_ref[...] = (jnp.dot(xm_ref[...], c3w0_ref[...],
                            preferred_element_type=_f32)
                    - dis_ref[...] * sy + b3_ref[...])


def _tcd(xm, c3w0, b3, dis, sy0, sy1):
    return pl.pallas_call(
        _tcd_body,
        grid=(_NBLK,),
        in_specs=[_rows(100), _full((100, 1)), _full((1, 1)), _rows(1),
                  _rows(1), _rows(1)],
        out_specs=_rows(1),
        out_shape=jax.ShapeDtypeStruct((_N, 1), _f32),
    )(xm, c3w0, b3, dis, sy0, sy1)


_LBLK = 6400
_LNBLK = _E // _LBLK


def _loss_body(p_ref, n_ref, out_ref, acc_ref):
    i = pl.program_id(0)

    @pl.when(i == 0)
    def _():
        acc_ref[0, 0] = 0.0

    sp = p_ref[0]
    sn = n_ref[0]
    term = (jnp.sum(jnp.log(jax.nn.sigmoid(sp) + 1e-15)) +
            jnp.sum(jnp.log(1.0 - jax.nn.sigmoid(sn))))
    acc_ref[0, 0] += term

    @pl.when(i == _LNBLK - 1)
    def _():
        out_ref[...] = jnp.full((1, 1), -acc_ref[0, 0] / float(_E), _f32)


def _loss(p, n):
    espec = pl.BlockSpec((1, _LBLK), lambda i: (0, i))
    return pl.pallas_call(
        _loss_body,
        grid=(_LNBLK,),
        in_specs=[espec, espec],
        out_specs=pl.BlockSpec((1, 1), lambda i: (0, 0)),
        out_shape=jax.ShapeDtypeStruct((1, 1), _f32),
        scratch_shapes=[pltpu.SMEM((1, 1), _f32)],
    )(p, n)


def kernel(x, edge_index, conv1_W0, conv1_W1, conv1_b, conv2_W0, conv2_W1,
           conv2_b, conv3_W0, conv3_W1, conv3_b, lin1_W, lin1_b, lin2_W,
           lin2_b, c1, c2):
    row = edge_index[0]
    col = edge_index[1]
    neg_ei = jax.random.randint(jax.random.key(1), (2, _E), 0, _N,
                                dtype=jnp.int32)

    deg0, deg1, colp = _sca(row, col)

    b1 = conv1_b.reshape(1, 300)
    l1b = lin1_b.reshape(1, 100)
    l2b = lin2_b.reshape(1, 100)
    xs0, xs1, xw0, l1, l2, dis = _tca(x, deg0[:_N, None], deg1[:_N, None],
                                      conv1_W0, lin1_W, l1b, lin2_W, l2b)

    (sx,) = _segsum2(row, colp, xs0, xs1)

    w1pad = jnp.concatenate([conv1_W1, jnp.zeros((6, 300), _f32)], axis=0)
    hw0, hs0, hs1, hs2, hs3 = _tcb(xw0, sx[:, :_N, :], dis, w1pad[:32],
                                   w1pad[32:], b1, conv2_W0, conv2_W1)

    (shw,) = _segsum4(row, colp, hs0, hs1, hs2, hs3)

    b2 = conv2_b.reshape(1, 100)
    xm, zt, ys = _tcc(hw0, shw[:, :_N, :], dis, b2, l1, l2, conv3_W1)

    posp, negp, sy0, sy1 = _scd(row, col, colp, neg_ei[0], neg_ei[1],
                                zt, ys.reshape(_N))

    b3 = conv3_b.reshape(1, 1)
    out = _tcd(xm, conv3_W0, b3, dis, sy0[:_N, None], sy1[:_N, None])
    r_loss = _loss(posp.reshape(1, _E), negp.reshape(1, _E)).reshape(())
    return (out, r_loss, c1, c2)


# speedup vs baseline: 1.1826x; 1.0010x over previous
import jax
import jax.numpy as jnp
from jax import lax
from jax.experimental import pallas as pl
from jax.experimental.pallas import tpu as pltpu
from jax.experimental.pallas import tpu_sc as plsc

_N = 50000
_E = 800000
_NPAD = 50176
_RPT = _NPAD // 16
_DUMMY = _N
_W = 2000
_NB = 25
_BB = _W // _NB
_NWIN = _E // (16 * _W)
_WSPLIT = 13

_mesh = plsc.VectorSubcoreMesh(
    core_axis_name="c", subcore_axis_name="s", num_cores=2, num_subcores=16)

_f32 = jnp.float32
_i32 = jnp.int32


def _iota16():
    return lax.broadcasted_iota(_i32, (16,), 0)


def _zero1d(buf, n):
    def z(i, carry):
        buf[pl.ds(i * 16, 16)] = jnp.zeros((16,), _f32)
        return carry
    lax.fori_loop(0, n // 16, z, 0)


def _zero2d(buf, n):
    def z(i, carry):
        buf[i, pl.ds(0, 16)] = jnp.zeros((16,), _f32)
        buf[i, pl.ds(16, 16)] = jnp.zeros((16,), _f32)
        return carry
    lax.fori_loop(0, n, z, 0)


def _sca_body(row_h, col_h, deg0_h, deg1_h, colp_h,
              acc, wbuf, rbuf, cbuf, rp80, cpbuf, ones):
    c = lax.axis_index("c")
    s = lax.axis_index("s")
    _zero1d(wbuf, _RPT)
    pltpu.sync_copy(wbuf, acc.at[pl.ds(s * _RPT, _RPT)])
    for q in range(_BB // 16):
        ones[pl.ds(q * 16, 16)] = jnp.ones((16,), _f32)
    plsc.subcore_barrier()

    wlo = jnp.where(c == 0, 0, _WSPLIT)
    whi = jnp.where(c == 0, _WSPLIT, _NWIN)

    def win(w, carry):
        ebase = s * (_E // 16) + w * _W
        pltpu.sync_copy(row_h.at[pl.ds(ebase, _W)], rbuf)
        pltpu.sync_copy(col_h.at[pl.ds(ebase, _W)], cbuf)

        def batch(j, carry2):
            for q in range(_BB // 16):
                off = j * _BB + q * 16
                rv = rbuf[pl.ds(off, 16)]
                cv = cbuf[pl.ds(off, 16)]
                keep = rv != cv
                rp80[pl.ds(q * 16, 16)] = jnp.where(keep, rv, _DUMMY)
                cpbuf[pl.ds(off, 16)] = jnp.where(keep, cv, _DUMMY)
            pltpu.sync_copy(ones, acc.at[rp80], add=True)
            return carry2
        lax.fori_loop(0, _NB, batch, 0)

        pltpu.sync_copy(cpbuf, colp_h.at[pl.ds(ebase, _W)])
        return carry
    lax.fori_loop(wlo, whi, win, 0)

    plsc.subcore_barrier()
    pltpu.sync_copy(acc.at[pl.ds(s * _RPT, _RPT)], wbuf)

    @pl.when(c == 0)
    def _():
        pltpu.sync_copy(wbuf, deg0_h.at[pl.ds(s * _RPT, _RPT)])

    @pl.when(c == 1)
    def _():
        pltpu.sync_copy(wbuf, deg1_h.at[pl.ds(s * _RPT, _RPT)])


def _sca(row, col):
    k = pl.kernel(
        _sca_body,
        out_type=[jax.ShapeDtypeStruct((_NPAD,), _f32),
                  jax.ShapeDtypeStruct((_NPAD,), _f32),
                  jax.ShapeDtypeStruct((_E,), _i32)],
        mesh=_mesh,
        compiler_params=pltpu.CompilerParams(use_tc_tiling_on_sc=False, needs_layout_passes=False, disable_bounds_checks=True),
        scratch_types=[
            pltpu.VMEM_SHARED((_NPAD,), _f32),
            pltpu.VMEM((_RPT,), _f32),
            pltpu.VMEM((_W,), _i32),
            pltpu.VMEM((_W,), _i32),
            pltpu.VMEM((_BB,), _i32),
            pltpu.VMEM((_W,), _i32),
            pltpu.VMEM((_BB,), _f32),
        ],
    )
    return k(row, col)


def _fill_idx(dst80, src, j):
    for q in range(_BB // 16):
        dst80[pl.ds(q * 16, 16)] = src[pl.ds(j * _BB + q * 16, 16)]


_WCH = _RPT // 8


def _segsum_pass(tab, ti, row_h, colp_h, out_h, acc, wbuf,
                 rbuf, cbuf, cia, cib, rows_a, rows_b,
                 sem_a, sem_b, s):
    _zero2d(wbuf, _WCH)
    for t in range(8):
        pltpu.sync_copy(wbuf, acc.at[pl.ds(s * _RPT + t * _WCH, _WCH)])
    plsc.subcore_barrier()

    def gat(j, dst, sem):
        return pltpu.make_async_copy(
            tab.at[rbuf.at[pl.ds(j * _BB, _BB)]], dst, sem)

    def win(w, carry):
        ebase = s * (_E // 16) + w * _W
        pltpu.sync_copy(row_h.at[pl.ds(ebase, _W)], rbuf)
        pltpu.sync_copy(colp_h.at[pl.ds(ebase, _W)], cbuf)

        gat(0, rows_a, sem_a).start()

        def pair(i, carry2):
            j = 2 * i
            gat(j, rows_a, sem_a).wait()
            gat(j + 1, rows_b, sem_b).start()
            _fill_idx(cia, cbuf, j)
            pltpu.sync_copy(rows_a, acc.at[cia], add=True)

            gat(j + 1, rows_b, sem_b).wait()
            gat(j + 2, rows_a, sem_a).start()
            _fill_idx(cib, cbuf, j + 1)
            pltpu.sync_copy(rows_b, acc.at[cib], add=True)
            return carry2
        lax.fori_loop(0, (_NB - 1) // 2, pair, 0)

        gat(_NB - 1, rows_a, sem_a).wait()
        _fill_idx(cia, cbuf, _NB - 1)
        pltpu.sync_copy(rows_a, acc.at[cia], add=True)
        return carry
    lax.fori_loop(0, _NWIN, win, 0)

    plsc.subcore_barrier()
    for t in range(8):
        pltpu.sync_copy(acc.at[pl.ds(s * _RPT + t * _WCH, _WCH)], wbuf)
        pltpu.sync_copy(wbuf, out_h.at[ti, pl.ds(s * _RPT + t * _WCH, _WCH)])


def _make_segsum(nt):
    tpc = nt // 2

    def body(*refs):
        row_h, colp_h = refs[0], refs[1]
        tabs = refs[2:2 + nt]
        out_h = refs[2 + nt]
        (acc, wbuf, rbuf, cbuf, cia, cib,
         rows_a, rows_b, sem_a, sem_b) = refs[3 + nt:]
        c = lax.axis_index("c")
        s = lax.axis_index("s")
        for cc in range(2):
            @pl.when(c == cc)
            def _():
                for j in range(tpc):
                    ti = cc * tpc + j
                    _segsum_pass(tabs[ti], ti, row_h, colp_h,
                                 out_h, acc, wbuf, rbuf, cbuf, cia, cib,
                                 rows_a, rows_b, sem_a, sem_b, s)

    def run(row, colp, *tables):
        k = pl.kernel(
            body,
            out_type=[jax.ShapeDtypeStruct((nt, _NPAD, 32), _f32)],
            mesh=_mesh,
            compiler_params=pltpu.CompilerParams(use_tc_tiling_on_sc=False, needs_layout_passes=False, disable_bounds_checks=True),
            scratch_types=[
                pltpu.VMEM_SHARED((_NPAD, 32), _f32),
                pltpu.VMEM((_WCH, 32), _f32),
                pltpu.VMEM((_W,), _i32),
                pltpu.VMEM((_W,), _i32),
                pltpu.VMEM((_BB,), _i32),
                pltpu.VMEM((_BB,), _i32),
                pltpu.VMEM((_BB, 32), _f32),
                pltpu.VMEM((_BB, 32), _f32),
                pltpu.SemaphoreType.DMA,
                pltpu.SemaphoreType.DMA,
            ],
        )
        return k(row, colp, *tables)
    return run


_segsum2 = _make_segsum(2)
_segsum4 = _make_segsum(4)


_DZ = 112


def _scd_body(row_h, col_h, colp_h, nrow_h, ncol_h, zt_h, ys_h,
              posp_h, negp_h, sy0_h, sy1_h,
              acc1, wbuf, rbuf, cbuf, cpbuf, cia, zra, zca, zrb, zcb,
              yrows, pbuf, sem_a, sem_b, sem_c, sem_d, sem_y):
    c = lax.axis_index("c")
    s = lax.axis_index("s")
    _zero1d(wbuf, _RPT)
    pltpu.sync_copy(wbuf, acc1.at[pl.ds(s * _RPT, _RPT)])
    plsc.subcore_barrier()

    wlo = jnp.where(c == 0, 0, _WSPLIT)
    whi = jnp.where(c == 0, _WSPLIT, _NWIN)

    def gz(idxbuf, j, dst, sem):
        return pltpu.make_async_copy(
            zt_h.at[idxbuf.at[pl.ds(j * _BB, _BB)]], dst, sem)

    def win(w, carry):
        ebase = s * (_E // 16) + w * _W
        pltpu.sync_copy(colp_h.at[pl.ds(ebase, _W)], cpbuf)

        def phase(p, carry1):
            @pl.when(p == 0)
            def _():
                pltpu.sync_copy(row_h.at[pl.ds(ebase, _W)], rbuf)
                pltpu.sync_copy(col_h.at[pl.ds(ebase, _W)], cbuf)

            @pl.when(p == 1)
            def _():
                pltpu.sync_copy(nrow_h.at[pl.ds(ebase, _W)], rbuf)
                pltpu.sync_copy(ncol_h.at[pl.ds(ebase, _W)], cbuf)

            def step(j, zr, zc, sr, sc2, nzr, nzc, nsr, nsc, issue_next):
                gz(rbuf, j, zr, sr).wait()
                gz(cbuf, j, zc, sc2).wait()
                if issue_next:
                    gz(rbuf, j + 1, nzr, nsr).start()
                    gz(cbuf, j + 1, nzc, nsc).start()

                @pl.when(p == 0)
                def _():
                    _fill_idx(cia, cpbuf, j)
                    pltpu.async_copy(
                        ys_h.at[rbuf.at[pl.ds(j * _BB, _BB)]],
                        yrows, sem_y).wait()
                    pltpu.sync_copy(yrows, acc1.at[cia], add=True)

                def chunk(k, carry2):
                    eidx = k * 16 + _iota16()
                    accv = jnp.zeros((16,), _f32)
                    for f in range(_DZ):
                        fidx = jnp.full((16,), f, _i32)
                        accv = accv + (plsc.load_gather(zr, [eidx, fidx]) *
                                       plsc.load_gather(zc, [eidx, fidx]))
                    pbuf[pl.ds(j * _BB + k * 16, 16)] = accv
                    return carry2
                lax.fori_loop(0, _BB // 16, chunk, 0)

            gz(rbuf, 0, zra, sem_a).start()
            gz(cbuf, 0, zca, sem_b).start()

            def pair(i, carry2):
                j = 2 * i
                step(j, zra, zca, sem_a, sem_b,
                     zrb, zcb, sem_c, sem_d, True)
                step(j + 1, zrb, zcb, sem_c, sem_d,
                     zra, zca, sem_a, sem_b, True)
                return carry2
            lax.fori_loop(0, (_NB - 1) // 2, pair, 0)
            step(_NB - 1, zra, zca, sem_a, sem_b, None, None, None, None,
                 False)

            @pl.when(p == 0)
            def _():
                pltpu.sync_copy(pbuf, posp_h.at[pl.ds(ebase, _W)])

            @pl.when(p == 1)
            def _():
                pltpu.sync_copy(pbuf, negp_h.at[pl.ds(ebase, _W)])
            return carry1
        lax.fori_loop(0, 2, phase, 0)
        return carry
    lax.fori_loop(wlo, whi, win, 0)

    plsc.subcore_barrier()
    pltpu.sync_copy(acc1.at[pl.ds(s * _RPT, _RPT)], wbuf)

    @pl.when(c == 0)
    def _():
        pltpu.sync_copy(wbuf, sy0_h.at[pl.ds(s * _RPT, _RPT)])

    @pl.when(c == 1)
    def _():
        pltpu.sync_copy(wbuf, sy1_h.at[pl.ds(s * _RPT, _RPT)])


def _scd(row, col, colp, nrow, ncol, zt, ys):
    k = pl.kernel(
        _scd_body,
        out_type=[jax.ShapeDtypeStruct((_E,), _f32),
                  jax.ShapeDtypeStruct((_E,), _f32),
                  jax.ShapeDtypeStruct((_NPAD,), _f32),
                  jax.ShapeDtypeStruct((_NPAD,), _f32)],
        mesh=_mesh,
        compiler_params=pltpu.CompilerParams(use_tc_tiling_on_sc=False, needs_layout_passes=False, disable_bounds_checks=True),
        scratch_types=[
            pltpu.VMEM_SHARED((_NPAD,), _f32),
            pltpu.VMEM((_RPT,), _f32),
            pltpu.VMEM((_W,), _i32),
            pltpu.VMEM((_W,), _i32),
            pltpu.VMEM((_W,), _i32),
            pltpu.VMEM((_BB,), _i32),
            pltpu.VMEM((_BB, _DZ), _f32),
            pltpu.VMEM((_BB, _DZ), _f32),
            pltpu.VMEM((_BB, _DZ), _f32),
            pltpu.VMEM((_BB, _DZ), _f32),
            pltpu.VMEM((_BB,), _f32),
            pltpu.VMEM((_W,), _f32),
            pltpu.SemaphoreType.DMA,
            pltpu.SemaphoreType.DMA,
            pltpu.SemaphoreType.DMA,
            pltpu.SemaphoreType.DMA,
            pltpu.SemaphoreType.DMA,
        ],
    )
    return k(row, col, colp, nrow, ncol, zt, ys)


_BLK = 1000
_NBLK = _N // _BLK


def _full(shape):
    return pl.BlockSpec(shape, lambda i: tuple(0 for _ in shape))


def _rows(width):
    return pl.BlockSpec((_BLK, width), lambda i: (i, 0))


def _tca_body(x_ref, d0_ref, d1_ref, w0_ref, l1w_ref, l1b_ref, l2w_ref,
              l2b_ref,
              xs0_ref, xs1_ref, xw0_ref, l1_ref, l2_ref, dis_ref):
    x = x_ref[...]
    deg = d0_ref[...] + d1_ref[...]
    dis = jnp.where(deg > 0.0, lax.rsqrt(jnp.maximum(deg, 1e-12)), 0.0)
    dis_ref[...] = dis
    xs = x * dis
    xs0_ref[...] = xs[:, :32]
    xs1_ref[...] = jnp.concatenate(
        [xs[:, 32:], jnp.zeros((_BLK, 6), _f32)], axis=1)
    xw0_ref[...] = jnp.dot(x, w0_ref[...], preferred_element_type=_f32)
    l1_ref[...] = jax.nn.relu(
        jnp.dot(x, l1w_ref[...], preferred_element_type=_f32) + l1b_ref[...])
    l2_ref[...] = jax.nn.relu(
        jnp.dot(x, l2w_ref[...], preferred_element_type=_f32) + l2b_ref[...])


def _tca(x, deg0, deg1, w0, l1w, l1b, l2w, l2b):
    return pl.pallas_call(
        _tca_body,
        grid=(_NBLK,),
        in_specs=[_rows(58), _rows(1), _rows(1),
                  _full((58, 300)), _full((58, 100)), _full((1, 100)),
                  _full((58, 100)), _full((1, 100))],
        out_specs=[_rows(32), _rows(32), _rows(300), _rows(100), _rows(100),
                   _rows(1)],
        out_shape=[jax.ShapeDtypeStruct((_N, 32), _f32),
                   jax.ShapeDtypeStruct((_N, 32), _f32),
                   jax.ShapeDtypeStruct((_N, 300), _f32),
                   jax.ShapeDtypeStruct((_N, 100), _f32),
                   jax.ShapeDtypeStruct((_N, 100), _f32),
                   jax.ShapeDtypeStruct((_N, 1), _f32)],
    )(x, deg0, deg1, w0, l1w, l1b, l2w, l2b)


def _tcb_body(xw0_ref, sx_ref, dis_ref, w1a_ref, w1b_ref, b1_ref,
              c2w0_ref, c2w1_ref,
              hw0_ref, hs0_ref, hs1_ref, hs2_ref, hs3_ref):
    sx = sx_ref[...]
    dis = dis_ref[...]
    tx1 = (jnp.dot(sx[0], w1a_ref[...], preferred_element_type=_f32) +
           jnp.dot(sx[1], w1b_ref[...], preferred_element_type=_f32))
    h = jax.nn.relu(xw0_ref[...] - dis * tx1 + b1_ref[...])
    hw0_ref[...] = jnp.dot(h, c2w0_ref[...], preferred_element_type=_f32)
    hs = dis * jnp.dot(h, c2w1_ref[...], preferred_element_type=_f32)
    hs0_ref[...] = hs[:, :32]
    hs1_ref[...] = hs[:, 32:64]
    hs2_ref[...] = hs[:, 64:96]
    hs3_ref[...] = jnp.concatenate(
        [hs[:, 96:], jnp.zeros((_BLK, 28), _f32)], axis=1)


def _tcb(xw0, sx, dis, w1a, w1b, b1, c2w0, c2w1):
    return pl.pallas_call(
        _tcb_body,
        grid=(_NBLK,),
        in_specs=[_rows(300),
                  pl.BlockSpec((2, _BLK, 32), lambda i: (0, i, 0)),
                  _rows(1),
                  _full((32, 300)), _full((32, 300)), _full((1, 300)),
                  _full((300, 100)), _full((300, 100))],
        out_specs=[_rows(100), _rows(32), _rows(32), _rows(32), _rows(32)],
        out_shape=[jax.ShapeDtypeStruct((_N, 100), _f32)] +
                  [jax.ShapeDtypeStruct((_N, 32), _f32)] * 4,
    )(xw0, sx, dis, w1a, w1b, b1, c2w0, c2w1)


def _tcc_body(hw0_ref, shw_ref, dis_ref, b2_ref, l1_ref, l2_ref, c3w1_ref,
              xm_ref, zt_ref, ys_ref):
    shw = shw_ref[...]
    dis = dis_ref[...]
    svals = jnp.concatenate([shw[0], shw[1], shw[2], shw[3]], axis=1)[:, :100]
    x1 = jax.nn.relu(hw0_ref[...] - dis * svals + b2_ref[...])
    xm = x1 + l1_ref[...]
    z = x1 + l2_ref[...]
    xm_ref[...] = xm
    zt_ref[...] = jnp.concatenate(
        [z, jnp.zeros((_BLK, _DZ - 100), _f32)], axis=1)
    ys_ref[...] = dis * jnp.dot(xm, c3w1_ref[...],
                                preferred_element_type=_f32)


def _tcc(hw0, shw, dis, b2, l1, l2, c3w1):
    return pl.pallas_call(
        _tcc_body,
        grid=(_NBLK,),
        in_specs=[_rows(100),
                  pl.BlockSpec((4, _BLK, 32), lambda i: (0, i, 0)),
                  _rows(1), _full((1, 100)), _rows(100), _rows(100),
                  _full((100, 1))],
        out_specs=[_rows(100), _rows(_DZ), _rows(1)],
        out_shape=[jax.ShapeDtypeStruct((_N, 100), _f32),
                   jax.ShapeDtypeStruct((_N, _DZ), _f32),
                   jax.ShapeDtypeStruct((_N, 1), _f32)],
    )(hw0, shw, dis, b2, l1, l2, c3w1)


def _tcd_body(xm_ref, c3w0_ref, b3_ref, dis_ref, sy0_ref, sy1_ref, out_ref):
    sy = sy0_ref[...] + sy1_ref[...]
    out_ref[...] = (jnp.dot(xm_ref[...], c3w0_ref[...],
                            preferred_element_type=_f32)
                    - dis_ref[...] * sy + b3_ref[...])


def _tcd(xm, c3w0, b3, dis, sy0, sy1):
    return pl.pallas_call(
        _tcd_body,
        grid=(_NBLK,),
        in_specs=[_rows(100), _full((100, 1)), _full((1, 1)), _rows(1),
                  _rows(1), _rows(1)],
        out_specs=_rows(1),
        out_shape=jax.ShapeDtypeStruct((_N, 1), _f32),
    )(xm, c3w0, b3, dis, sy0, sy1)


_LBLK = 6400
_LNBLK = _E // _LBLK


def _loss_body(p_ref, n_ref, out_ref, acc_ref):
    i = pl.program_id(0)

    @pl.when(i == 0)
    def _():
        acc_ref[0, 0] = 0.0

    sp = p_ref[0]
    sn = n_ref[0]
    term = (jnp.sum(jnp.log(jax.nn.sigmoid(sp) + 1e-15)) +
            jnp.sum(jnp.log(1.0 - jax.nn.sigmoid(sn))))
    acc_ref[0, 0] += term

    @pl.when(i == _LNBLK - 1)
    def _():
        out_ref[...] = jnp.full((1, 1), -acc_ref[0, 0] / float(_E), _f32)


def _loss(p, n):
    espec = pl.BlockSpec((1, _LBLK), lambda i: (0, i))
    return pl.pallas_call(
        _loss_body,
        grid=(_LNBLK,),
        in_specs=[espec, espec],
        out_specs=pl.BlockSpec((1, 1), lambda i: (0, 0)),
        out_shape=jax.ShapeDtypeStruct((1, 1), _f32),
        scratch_shapes=[pltpu.SMEM((1, 1), _f32)],
    )(p, n)


def kernel(x, edge_index, conv1_W0, conv1_W1, conv1_b, conv2_W0, conv2_W1,
           conv2_b, conv3_W0, conv3_W1, conv3_b, lin1_W, lin1_b, lin2_W,
           lin2_b, c1, c2):
    row = edge_index[0]
    col = edge_index[1]
    neg_ei = jax.random.randint(jax.random.key(1), (2, _E), 0, _N,
                                dtype=jnp.int32)

    deg0, deg1, colp = _sca(row, col)

    b1 = conv1_b.reshape(1, 300)
    l1b = lin1_b.reshape(1, 100)
    l2b = lin2_b.reshape(1, 100)
    xs0, xs1, xw0, l1, l2, dis = _tca(x, deg0[:_N, None], deg1[:_N, None],
                                      conv1_W0, lin1_W, l1b, lin2_W, l2b)

    (sx,) = _segsum2(row, colp, xs0, xs1)

    w1pad = jnp.concatenate([conv1_W1, jnp.zeros((6, 300), _f32)], axis=0)
    hw0, hs0, hs1, hs2, hs3 = _tcb(xw0, sx[:, :_N, :], dis, w1pad[:32],
                                   w1pad[32:], b1, conv2_W0, conv2_W1)

    (shw,) = _segsum4(row, colp, hs0, hs1, hs2, hs3)

    b2 = conv2_b.reshape(1, 100)
    xm, zt, ys = _tcc(hw0, shw[:, :_N, :], dis, b2, l1, l2, conv3_W1)

    posp, negp, sy0, sy1 = _scd(row, col, colp, neg_ei[0], neg_ei[1],
                                zt, ys.reshape(_N))

    b3 = conv3_b.reshape(1, 1)
    out = _tcd(xm, conv3_W0, b3, dis, sy0[:_N, None], sy1[:_N, None])
    r_loss = _loss(posp.reshape(1, _E), negp.reshape(1, _E)).reshape(())
    return (out, r_loss, c1, c2)
